# Initial kernel scaffold; baseline (speedup 1.0000x reference)
import jax
import jax.numpy as jnp
from jax.experimental import pallas as pl


def kernel(feat, edge_index, edge_attr, W, b):
    raise NotImplementedError("write your pallas kernel here")


# speedup vs baseline: 3.3967x; 3.3967x over previous
import functools

import jax
import jax.numpy as jnp
from jax import lax
from jax.experimental import pallas as pl
from jax.experimental.pallas import tpu as pltpu
from jax.experimental.pallas import tpu_sc as plsc

N_NODES = 10000
N_EDGES = 320000
D_FEAT = 128
D_EDGE = 16

NC = 2
NS = 16
NW = NC * NS
EB = 128
NBLK = N_EDGES // EB
BLK_PER_W = NBLK // NW
BLK_EXTRA = NBLK - BLK_PER_W * NW
RPT = 624
RPT_LAST = N_NODES - (NS - 1) * RPT


def _node_tables_body(feat_ref, w_ref, p1_ref, p2_ref):
    f = feat_ref[...]
    p1_ref[...] = jnp.dot(f, w_ref[0:D_FEAT, :], preferred_element_type=jnp.float32)
    p2_ref[...] = jnp.dot(f, w_ref[D_FEAT:2 * D_FEAT, :], preferred_element_type=jnp.float32)


def _node_tables(feat, W):
    return pl.pallas_call(
        _node_tables_body,
        out_shape=(
            jax.ShapeDtypeStruct((N_NODES, D_FEAT), jnp.float32),
            jax.ShapeDtypeStruct((N_NODES, D_FEAT), jnp.float32),
        ),
    )(feat, W)


_CEB = 8000


def _edge_c_body(ea_ref, w_ref, b_ref, c_ref):
    c_ref[...] = (
        jnp.dot(ea_ref[...], w_ref[2 * D_FEAT:, :], preferred_element_type=jnp.float32)
        + b_ref[...]
    )


def _edge_c(edge_attr, W, b2d):
    n_steps = N_EDGES // _CEB
    return pl.pallas_call(
        _edge_c_body,
        grid=(n_steps,),
        in_specs=[
            pl.BlockSpec((_CEB, D_EDGE), lambda i: (i, 0)),
            pl.BlockSpec((2 * D_FEAT + D_EDGE, D_FEAT), lambda i: (0, 0)),
            pl.BlockSpec((1, D_FEAT), lambda i: (0, 0)),
        ],
        out_specs=pl.BlockSpec((_CEB, D_FEAT), lambda i: (i, 0)),
        out_shape=jax.ShapeDtypeStruct((N_EDGES, D_FEAT), jnp.float32),
    )(edge_attr, W, b2d)


def _sc_edge_body(p1_hbm, p2_hbm, c_hbm, src_hbm, dst_hbm, zeros_hbm, out_hbm,
                  sidx, didx, p1v, p2v, cv, acc, sem1, sem2, sem3):
    cid = lax.axis_index("c")
    sid = lax.axis_index("s")
    wid = cid * NS + sid

    r0 = sid * RPT
    pltpu.sync_copy(zeros_hbm.at[pl.ds(0, RPT)], acc.at[pl.ds(r0, RPT)])

    @pl.when(sid == NS - 1)
    def _zero_tail():
        pltpu.sync_copy(
            zeros_hbm.at[pl.ds(0, RPT_LAST - RPT)],
            acc.at[pl.ds((NS - 1) * RPT + RPT, RPT_LAST - RPT)],
        )

    plsc.subcore_barrier()

    nblk = jnp.where(wid < BLK_EXTRA, BLK_PER_W + 1, BLK_PER_W)
    bstart = wid * BLK_PER_W + jnp.minimum(wid, BLK_EXTRA)

    def block_body(i, carry):
        e0 = (bstart + i) * EB
        pltpu.sync_copy(src_hbm.at[pl.ds(e0, EB)], sidx)
        pltpu.sync_copy(dst_hbm.at[pl.ds(e0, EB)], didx)
        cp1 = pltpu.async_copy(p1_hbm.at[sidx], p1v, sem1)
        cp2 = pltpu.async_copy(p2_hbm.at[didx], p2v, sem2)
        cp3 = pltpu.async_copy(c_hbm.at[pl.ds(e0, EB)], cv, sem3)
        cp1.wait()
        cp2.wait()
        cp3.wait()

        def row_body(r, c2):
            for k in range(D_FEAT // 16):
                sl = pl.ds(16 * k, 16)
                z = p1v[r, sl] + p2v[r, sl] + cv[r, sl]
                s = 1.0 / (1.0 + jnp.exp(-z))
                m = jnp.where(z >= 0.0, z, 0.01 * z)
                p1v[r, sl] = s * m
            return c2

        lax.fori_loop(0, EB, row_body, 0)
        pltpu.sync_copy(p1v, acc.at[didx], add=True)
        return carry

    lax.fori_loop(0, nblk, block_body, 0)

    plsc.subcore_barrier()
    pltpu.sync_copy(
        acc.at[pl.ds(r0, RPT)],
        out_hbm.at[pl.ds(cid * N_NODES + r0, RPT)],
    )

    @pl.when(sid == NS - 1)
    def _write_tail():
        t0 = (NS - 1) * RPT + RPT
        pltpu.sync_copy(
            acc.at[pl.ds(t0, RPT_LAST - RPT)],
            out_hbm.at[pl.ds(cid * N_NODES + t0, RPT_LAST - RPT)],
        )


def _sc_edge(p1, p2, c, src, dst, zeros):
    mesh = plsc.VectorSubcoreMesh(core_axis_name="c", subcore_axis_name="s")
    k = functools.partial(
        pl.kernel,
        out_type=jax.ShapeDtypeStruct((NC * N_NODES, D_FEAT), jnp.float32),
        mesh=mesh,
        scratch_types=[
            pltpu.VMEM((EB,), jnp.int32),
            pltpu.VMEM((EB,), jnp.int32),
            pltpu.VMEM((EB, D_FEAT), jnp.float32),
            pltpu.VMEM((EB, D_FEAT), jnp.float32),
            pltpu.VMEM((EB, D_FEAT), jnp.float32),
            pltpu.VMEM_SHARED((N_NODES, D_FEAT), jnp.float32),
            pltpu.SemaphoreType.DMA,
            pltpu.SemaphoreType.DMA,
            pltpu.SemaphoreType.DMA,
        ],
    )(_sc_edge_body)
    return k(p1, p2, c, src, dst, zeros)


_AB = 80


def _add_body(a_ref, b_ref, o_ref):
    o_ref[...] = a_ref[...] + b_ref[...]


def _final_add(part):
    n_steps = N_NODES // _AB
    return pl.pallas_call(
        _add_body,
        grid=(n_steps,),
        in_specs=[
            pl.BlockSpec((_AB, D_FEAT), lambda i: (i, 0)),
            pl.BlockSpec((_AB, D_FEAT), lambda i: (i + N_NODES // _AB, 0)),
        ],
        out_specs=pl.BlockSpec((_AB, D_FEAT), lambda i: (i, 0)),
        out_shape=jax.ShapeDtypeStruct((N_NODES, D_FEAT), jnp.float32),
    )(part, part)


def kernel(feat, edge_index, edge_attr, W, b):
    src = edge_index[0].astype(jnp.int32)
    dst = edge_index[1].astype(jnp.int32)
    p1, p2 = _node_tables(feat, W)
    c = _edge_c(edge_attr, W, b.reshape(1, D_FEAT))
    zeros = jnp.zeros((RPT, D_FEAT), jnp.float32)
    part = _sc_edge(p1, p2, c, src, dst, zeros)
    return _final_add(part)


# speedup vs baseline: 4.3176x; 1.2711x over previous
import functools

import jax
import jax.numpy as jnp
from jax import lax
from jax.experimental import pallas as pl
from jax.experimental.pallas import tpu as pltpu
from jax.experimental.pallas import tpu_sc as plsc

N_NODES = 10000
N_EDGES = 320000
D_FEAT = 128
D_EDGE = 16

NC = 2
NS = 16
NW = NC * NS
EB = 64
NBLK = N_EDGES // EB
BLK_PER_W = NBLK // NW
BLK_EXTRA = NBLK - BLK_PER_W * NW
MAXB = BLK_PER_W + 1
RPT = 624
RPT_LAST = N_NODES - (NS - 1) * RPT


def _node_tables_body(feat_ref, w_ref, p1_ref, p2_ref):
    f = feat_ref[...]
    p1_ref[...] = jnp.dot(f, w_ref[0:D_FEAT, :], preferred_element_type=jnp.float32)
    p2_ref[...] = jnp.dot(f, w_ref[D_FEAT:2 * D_FEAT, :], preferred_element_type=jnp.float32)


def _node_tables(feat, W):
    return pl.pallas_call(
        _node_tables_body,
        out_shape=(
            jax.ShapeDtypeStruct((N_NODES, D_FEAT), jnp.float32),
            jax.ShapeDtypeStruct((N_NODES, D_FEAT), jnp.float32),
        ),
    )(feat, W)


_CEB = 8000


def _edge_c_body(ea_ref, w_ref, b_ref, c_ref):
    c_ref[...] = (
        jnp.dot(ea_ref[...], w_ref[2 * D_FEAT:, :], preferred_element_type=jnp.float32)
        + b_ref[...]
    )


def _edge_c(edge_attr, W, b2d):
    n_steps = N_EDGES // _CEB
    return pl.pallas_call(
        _edge_c_body,
        grid=(n_steps,),
        in_specs=[
            pl.BlockSpec((_CEB, D_EDGE), lambda i: (i, 0)),
            pl.BlockSpec((2 * D_FEAT + D_EDGE, D_FEAT), lambda i: (0, 0)),
            pl.BlockSpec((1, D_FEAT), lambda i: (0, 0)),
        ],
        out_specs=pl.BlockSpec((_CEB, D_FEAT), lambda i: (i, 0)),
        out_shape=jax.ShapeDtypeStruct((N_EDGES, D_FEAT), jnp.float32),
    )(edge_attr, W, b2d)


def _sc_edge_body(p1_hbm, p2_hbm, c_hbm, src_hbm, dst_hbm, zeros_hbm, out_hbm,
                  sidx0, sidx1, didx0, didx1, p1v0, p1v1, p2v0, p2v1,
                  cv0, cv1, acc, g1a, g1b, g2a, g2b, g3a, g3b, isa, isb):
    sidx = (sidx0, sidx1)
    didx = (didx0, didx1)
    p1v = (p1v0, p1v1)
    p2v = (p2v0, p2v1)
    cv = (cv0, cv1)
    g1 = (g1a, g1b)
    g2 = (g2a, g2b)
    g3 = (g3a, g3b)
    isem = (isa, isb)
    cid = lax.axis_index("c")
    sid = lax.axis_index("s")
    wid = cid * NS + sid

    r0 = sid * RPT
    pltpu.sync_copy(zeros_hbm.at[pl.ds(0, RPT)], acc.at[pl.ds(r0, RPT)])

    @pl.when(sid == NS - 1)
    def _zero_tail():
        pltpu.sync_copy(
            zeros_hbm.at[pl.ds(0, RPT_LAST - RPT)],
            acc.at[pl.ds((NS - 1) * RPT + RPT, RPT_LAST - RPT)],
        )

    nblk = jnp.where(wid < BLK_EXTRA, BLK_PER_W + 1, BLK_PER_W)
    bstart = wid * BLK_PER_W + jnp.minimum(wid, BLK_EXTRA)

    def issue_idx(i, b):
        e0 = (bstart + i) * EB
        pltpu.async_copy(src_hbm.at[pl.ds(e0, EB)], sidx[b], isem[b])
        pltpu.async_copy(dst_hbm.at[pl.ds(e0, EB)], didx[b], isem[b])

    def wait_idx(i, b):
        e0 = (bstart + i) * EB
        pltpu.make_async_copy(src_hbm.at[pl.ds(e0, EB)], sidx[b], isem[b]).wait()
        pltpu.make_async_copy(dst_hbm.at[pl.ds(e0, EB)], didx[b], isem[b]).wait()

    def issue_gathers(i, b):
        e0 = (bstart + i) * EB
        pltpu.async_copy(p1_hbm.at[sidx[b]], p1v[b], g1[b])
        pltpu.async_copy(p2_hbm.at[didx[b]], p2v[b], g2[b])
        pltpu.async_copy(c_hbm.at[pl.ds(e0, EB)], cv[b], g3[b])

    def wait_gathers(i, b):
        e0 = (bstart + i) * EB
        pltpu.make_async_copy(p1_hbm.at[sidx[b]], p1v[b], g1[b]).wait()
        pltpu.make_async_copy(p2_hbm.at[didx[b]], p2v[b], g2[b]).wait()
        pltpu.make_async_copy(c_hbm.at[pl.ds(e0, EB)], cv[b], g3[b]).wait()

    e00 = bstart * EB
    pltpu.sync_copy(src_hbm.at[pl.ds(e00, EB)], sidx[0])
    pltpu.sync_copy(dst_hbm.at[pl.ds(e00, EB)], didx[0])
    issue_gathers(0, 0)
    issue_idx(1, 1)

    def pair_body(j, carry):
        for b in (0, 1):
            blk = 2 * j + b
            nb = 1 - b

            @pl.when(blk < nblk)
            def _live():
                wait_gathers(blk, b)

                @pl.when(blk + 1 < nblk)
                def _prefetch():
                    wait_idx(blk + 1, nb)
                    issue_gathers(blk + 1, nb)

                def row_body(r, c2):
                    for k in range(D_FEAT // 16):
                        sl = pl.ds(16 * k, 16)
                        z = p1v[b][r, sl] + p2v[b][r, sl] + cv[b][r, sl]
                        m = jnp.maximum(z, 0.01 * z)
                        p1v[b][r, sl] = m / (1.0 + jnp.exp(-z))
                    return c2

                lax.fori_loop(0, EB, row_body, 0)
                pltpu.sync_copy(p1v[b], acc.at[didx[b]], add=True)

                @pl.when(blk + 2 < nblk)
                def _next_idx():
                    issue_idx(blk + 2, b)

        return carry

    lax.fori_loop(0, (MAXB + 1) // 2, pair_body, 0)

    plsc.subcore_barrier()
    pltpu.sync_copy(
        acc.at[pl.ds(r0, RPT)],
        out_hbm.at[pl.ds(cid * N_NODES + r0, RPT)],
    )

    @pl.when(sid == NS - 1)
    def _write_tail():
        t0 = (NS - 1) * RPT + RPT
        pltpu.sync_copy(
            acc.at[pl.ds(t0, RPT_LAST - RPT)],
            out_hbm.at[pl.ds(cid * N_NODES + t0, RPT_LAST - RPT)],
        )


def _sc_edge(p1, p2, c, src, dst, zeros):
    mesh = plsc.VectorSubcoreMesh(core_axis_name="c", subcore_axis_name="s")
    k = functools.partial(
        pl.kernel,
        out_type=jax.ShapeDtypeStruct((NC * N_NODES, D_FEAT), jnp.float32),
        mesh=mesh,
        scratch_types=[
            pltpu.VMEM((EB,), jnp.int32),
            pltpu.VMEM((EB,), jnp.int32),
            pltpu.VMEM((EB,), jnp.int32),
            pltpu.VMEM((EB,), jnp.int32),
            pltpu.VMEM((EB, D_FEAT), jnp.float32),
            pltpu.VMEM((EB, D_FEAT), jnp.float32),
            pltpu.VMEM((EB, D_FEAT), jnp.float32),
            pltpu.VMEM((EB, D_FEAT), jnp.float32),
            pltpu.VMEM((EB, D_FEAT), jnp.float32),
            pltpu.VMEM((EB, D_FEAT), jnp.float32),
            pltpu.VMEM_SHARED((N_NODES, D_FEAT), jnp.float32),
            pltpu.SemaphoreType.DMA,
            pltpu.SemaphoreType.DMA,
            pltpu.SemaphoreType.DMA,
            pltpu.SemaphoreType.DMA,
            pltpu.SemaphoreType.DMA,
            pltpu.SemaphoreType.DMA,
            pltpu.SemaphoreType.DMA,
            pltpu.SemaphoreType.DMA,
        ],
    )(_sc_edge_body)
    return k(p1, p2, c, src, dst, zeros)


_AB = 80


def _add_body(a_ref, b_ref, o_ref):
    o_ref[...] = a_ref[...] + b_ref[...]


def _final_add(part):
    n_steps = N_NODES // _AB
    return pl.pallas_call(
        _add_body,
        grid=(n_steps,),
        in_specs=[
            pl.BlockSpec((_AB, D_FEAT), lambda i: (i, 0)),
            pl.BlockSpec((_AB, D_FEAT), lambda i: (i + N_NODES // _AB, 0)),
        ],
        out_specs=pl.BlockSpec((_AB, D_FEAT), lambda i: (i, 0)),
        out_shape=jax.ShapeDtypeStruct((N_NODES, D_FEAT), jnp.float32),
    )(part, part)


def kernel(feat, edge_index, edge_attr, W, b):
    src = edge_index[0].astype(jnp.int32)
    dst = edge_index[1].astype(jnp.int32)
    p1, p2 = _node_tables(feat, W)
    c = _edge_c(edge_attr, W, b.reshape(1, D_FEAT))
    zeros = jnp.zeros((RPT, D_FEAT), jnp.float32)
    part = _sc_edge(p1, p2, c, src, dst, zeros)
    return _final_add(part)
